# baseline (speedup 1.0000x reference)
import functools

import jax
import jax.numpy as jnp
from jax import lax
from jax.experimental import pallas as pl
from jax.experimental.pallas import tpu as pltpu
from jax.experimental.pallas import tpu_sc as plsc

_B = 16384
_EMB = 32
_CH = 128


def _pad128(flat):
    n = flat.shape[0]
    pad = (-n) % 128
    return jnp.pad(flat, (0, pad)).reshape(-1, 128)


def _extract(chunk_ref, offs_v, out_v, row0, lanes):
    def egroup(g, _):
        rvec = lanes + g * 16
        o16 = offs_v[pl.ds(row0 + g * 16, 16)]
        for c in range(_EMB):
            vals = plsc.load_gather(chunk_ref, [rvec, o16 + c])
            plsc.store_scatter(out_v, [rvec + row0, lanes * 0 + c], vals)
        return ()

    lax.fori_loop(0, _CH // 16, egroup, (), unroll=1)


def _sc_gather(sup_u, sup_p, off_u, off_p, utp, ptp):
    info = plsc.get_sparse_core_info()
    nw = info.num_cores * info.num_subcores
    b_per_w = _B // nw
    mesh = plsc.VectorSubcoreMesh(core_axis_name="c", subcore_axis_name="s")

    @functools.partial(
        pl.kernel,
        mesh=mesh,
        compiler_params=pltpu.CompilerParams(
            use_tc_tiling_on_sc=True, needs_layout_passes=False),
        out_type=(
            jax.ShapeDtypeStruct((_B, _EMB), jnp.float32),
            jax.ShapeDtypeStruct((_B, _EMB), jnp.float32),
        ),
    scratch_types=[
            pltpu.VMEM((b_per_w,), jnp.int32),
            pltpu.VMEM((b_per_w,), jnp.int32),
            pltpu.VMEM((2, _CH, 128), jnp.float32),
            pltpu.VMEM((b_per_w, _EMB), jnp.float32),
            pltpu.SemaphoreType.DMA,
        ],
    )
    def gather_k(su_hbm, sp_hbm, ou_hbm, op_hbm, utp_hbm, ptp_hbm,
                 uout_hbm, pout_hbm, sidx_v, offs_v, rows_v, out_v, sem):
        wid = lax.axis_index("s") * info.num_cores + lax.axis_index("c")
        base = wid * b_per_w
        lanes = lax.iota(jnp.int32, 16)
        nch = b_per_w // _CH
        for (s_hbm, o_hbm, dst_hbm, tab) in (
                (su_hbm, ou_hbm, uout_hbm, utp_hbm),
                (sp_hbm, op_hbm, pout_hbm, ptp_hbm)):
            pltpu.sync_copy(s_hbm.at[pl.ds(base, b_per_w)], sidx_v)
            pltpu.sync_copy(o_hbm.at[pl.ds(base, b_per_w)], offs_v)
            copies = []
            for j in range(nch):
                sl = pl.ds(j * _CH, _CH)
                copies.append(pltpu.async_copy(
                    tab.at[sidx_v.at[sl]], rows_v.at[j % 2], sem))
                if j > 0:
                    copies[j - 1].wait()
                    _extract(rows_v.at[(j - 1) % 2], offs_v, out_v,
                             (j - 1) * _CH, lanes)
            copies[nch - 1].wait()
            _extract(rows_v.at[(nch - 1) % 2], offs_v, out_v,
                     (nch - 1) * _CH, lanes)
            pltpu.sync_copy(out_v, dst_hbm.at[pl.ds(base, b_per_w)])

    return gather_k(sup_u, sup_p, off_u, off_p, utp, ptp)


def _mlp_body(u_ref, p_ref, w1u_ref, w1p_ref, b1_ref, w2_ref, b2_ref,
              w3_ref, b3_ref, out_ref):
    h = u_ref[...] @ w1u_ref[...] + p_ref[...] @ w1p_ref[...] + b1_ref[...]
    h = jnp.maximum(h, 0.0)
    h = jnp.maximum(h @ w2_ref[...] + b2_ref[...], 0.0)
    out_ref[...] = h @ w3_ref[...] + b3_ref[...]


def _mlp(u_emb, p_emb, W1, b1, W2, b2, W3, b3):
    bb = 2048
    grid = (_B // bb,)
    return pl.pallas_call(
        _mlp_body,
        grid=grid,
        in_specs=[
            pl.BlockSpec((bb, _EMB), lambda i: (i, 0)),
            pl.BlockSpec((bb, _EMB), lambda i: (i, 0)),
            pl.BlockSpec((_EMB, 256), lambda i: (0, 0)),
            pl.BlockSpec((_EMB, 256), lambda i: (0, 0)),
            pl.BlockSpec((1, 256), lambda i: (0, 0)),
            pl.BlockSpec((256, 128), lambda i: (0, 0)),
            pl.BlockSpec((1, 128), lambda i: (0, 0)),
            pl.BlockSpec((128, 1), lambda i: (0, 0)),
            pl.BlockSpec((1, 1), lambda i: (0, 0)),
        ],
        out_specs=pl.BlockSpec((bb, 1), lambda i: (i, 0)),
        out_shape=jax.ShapeDtypeStruct((_B, 1), jnp.float32),
    )(u_emb, p_emb, W1[:_EMB], W1[_EMB:], b1.reshape(1, 256), W2,
      b2.reshape(1, 128), W3, b3.reshape(1, 1))


def kernel(user_id, product_id, user_table, product_table,
           W1, b1, W2, b2, W3, b3):
    utp = _pad128(user_table.reshape(-1))
    ptp = _pad128(product_table.reshape(-1))
    sup_u = user_id // 4
    sup_p = product_id // 4
    off_u = 32 * (user_id % 4)
    off_p = 32 * (product_id % 4)
    u_emb, p_emb = _sc_gather(sup_u, sup_p, off_u, off_p, utp, ptp)
    rating = _mlp(u_emb, p_emb, W1, b1, W2, b2, W3, b3)
    return (u_emb, p_emb, rating)


# speedup vs baseline: 1.5837x; 1.5837x over previous
import functools

import jax
import jax.numpy as jnp
from jax import lax
from jax.experimental import pallas as pl
from jax.experimental.pallas import tpu as pltpu
from jax.experimental.pallas import tpu_sc as plsc

_B = 16384
_EMB = 32
_WIN = 32


def _sc_gather(user_id, product_id, utab, ptab):
    info = plsc.get_sparse_core_info()
    nw = info.num_cores * info.num_subcores
    b_per_w = _B // nw
    mesh = plsc.VectorSubcoreMesh(core_axis_name="c", subcore_axis_name="s")

    @functools.partial(
        pl.kernel,
        mesh=mesh,
        compiler_params=pltpu.CompilerParams(
            use_tc_tiling_on_sc=True, needs_layout_passes=False,
            disable_bounds_checks=True),
        out_type=(
            jax.ShapeDtypeStruct((_B, _EMB), jnp.float32),
            jax.ShapeDtypeStruct((_B, _EMB), jnp.float32),
        ),
        scratch_types=[
            pltpu.VMEM((b_per_w,), jnp.int32),
            pltpu.VMEM((_WIN, 8, _EMB), jnp.float32),
            pltpu.VMEM((b_per_w, _EMB), jnp.float32),
            pltpu.SemaphoreType.DMA,
        ],
    )
    def gather_k(uid_hbm, pid_hbm, utab_hbm, ptab_hbm, uout_hbm, pout_hbm,
                 idx_v, win_v, out_v, sem):
        wid = lax.axis_index("s") * info.num_cores + lax.axis_index("c")
        base = wid * b_per_w
        lanes = lax.iota(jnp.int32, 16)
        for (id_hbm, tab, dst_hbm) in ((uid_hbm, utab_hbm, uout_hbm),
                                       (pid_hbm, ptab_hbm, pout_hbm)):
            pltpu.sync_copy(id_hbm.at[pl.ds(base, b_per_w)], idx_v)

            def group(t, _):
                k0 = t * _WIN
                vecs = [idx_v[pl.ds(k0 + h * 16, 16)]
                        for h in range(_WIN // 16)]
                copies = []
                for q in range(_WIN):
                    j = vecs[q // 16][q % 16]
                    ws = pl.multiple_of(8 * (j // 8), 8)
                    copies.append(pltpu.async_copy(
                        tab.at[pl.ds(ws, 8)], win_v.at[q], sem))
                for c in copies:
                    c.wait()
                for q in range(_WIN):
                    j = vecs[q // 16][q % 16]
                    r = lanes * 0 + (j % 8)
                    kv = lanes * 0 + (k0 + q)
                    lo = plsc.load_gather(win_v.at[q], [r, lanes])
                    hi = plsc.load_gather(win_v.at[q], [r, lanes + 16])
                    plsc.store_scatter(out_v, [kv, lanes], lo)
                    plsc.store_scatter(out_v, [kv, lanes + 16], hi)
                return ()

            lax.fori_loop(0, b_per_w // _WIN, group, (), unroll=1)
            pltpu.sync_copy(out_v, dst_hbm.at[pl.ds(base, b_per_w)])

    return gather_k(user_id, product_id, utab, ptab)


def _mlp_body(u_ref, p_ref, w1u_ref, w1p_ref, b1_ref, w2_ref, b2_ref,
              w3_ref, b3_ref, out_ref):
    h = u_ref[...] @ w1u_ref[...] + p_ref[...] @ w1p_ref[...] + b1_ref[...]
    h = jnp.maximum(h, 0.0)
    h = jnp.maximum(h @ w2_ref[...] + b2_ref[...], 0.0)
    out_ref[...] = h @ w3_ref[...] + b3_ref[...]


def _mlp(u_emb, p_emb, W1, b1, W2, b2, W3, b3):
    bb = 2048
    grid = (_B // bb,)
    return pl.pallas_call(
        _mlp_body,
        grid=grid,
        in_specs=[
            pl.BlockSpec((bb, _EMB), lambda i: (i, 0)),
            pl.BlockSpec((bb, _EMB), lambda i: (i, 0)),
            pl.BlockSpec((_EMB, 256), lambda i: (0, 0)),
            pl.BlockSpec((_EMB, 256), lambda i: (0, 0)),
            pl.BlockSpec((1, 256), lambda i: (0, 0)),
            pl.BlockSpec((256, 128), lambda i: (0, 0)),
            pl.BlockSpec((1, 128), lambda i: (0, 0)),
            pl.BlockSpec((128, 1), lambda i: (0, 0)),
            pl.BlockSpec((1, 1), lambda i: (0, 0)),
        ],
        out_specs=pl.BlockSpec((bb, 1), lambda i: (i, 0)),
        out_shape=jax.ShapeDtypeStruct((_B, 1), jnp.float32),
    )(u_emb, p_emb, W1[:_EMB], W1[_EMB:], b1.reshape(1, 256), W2,
      b2.reshape(1, 128), W3, b3.reshape(1, 1))


def kernel(user_id, product_id, user_table, product_table,
           W1, b1, W2, b2, W3, b3):
    u_emb, p_emb = _sc_gather(user_id, product_id, user_table, product_table)
    rating = _mlp(u_emb, p_emb, W1, b1, W2, b2, W3, b3)
    return (u_emb, p_emb, rating)
